# baseline (speedup 1.0000x reference)
import functools

import jax
import jax.numpy as jnp
import numpy as np
from jax.experimental import pallas as pl
from jax.experimental.pallas import tpu as pltpu

_ANCHORS = ((0.28, 0.22), (0.38, 0.48), (0.9, 0.78))
_NA = 3
_BPB = 4


def _body(ecls, exy, ewh, x_ref, boxes_ref, conf_ref, cls_ref, *, W, aw, ah):
    s = x_ref[...]
    P = s.shape[3]
    sig = 1.0 / (1.0 + jnp.exp(-s))

    t_cls = jax.lax.dot_general(
        sig, ecls, (((2,), (0,)), ((), ())),
        preferred_element_type=jnp.float32)
    cls_ref[...] = t_cls.reshape(cls_ref.shape)

    conf_ref[...] = sig[:, :, 4:5, :]

    lan = jax.lax.broadcasted_iota(jnp.int32, (2, P), 1)
    rid = jax.lax.broadcasted_iota(jnp.int32, (2, P), 0)
    off = jnp.where(rid == 0, lan // W, lan % W).astype(jnp.float32)
    xy = sig[:, :, 0:2, :] + off[None, None]
    ex = jnp.exp(s[:, :, 2:4, :])
    aid = jax.lax.broadcasted_iota(jnp.int32, ex.shape, 1)
    aw_v = jnp.where(aid == 0, aw[0], jnp.where(aid == 1, aw[1], aw[2]))
    ah_v = jnp.where(aid == 0, ah[0], jnp.where(aid == 1, ah[1], ah[2]))
    rid2 = jax.lax.broadcasted_iota(jnp.int32, ex.shape, 2)
    wh = ex * jnp.where(rid2 == 0, aw_v, ah_v)
    t_box = (
        jax.lax.dot_general(xy, exy, (((2,), (0,)), ((), ())),
                            preferred_element_type=jnp.float32)
        + jax.lax.dot_general(wh, ewh, (((2,), (0,)), ((), ())),
                              preferred_element_type=jnp.float32))
    boxes_ref[...] = t_box.reshape(boxes_ref.shape)


def _outer(x_hbm, ecls_ref, exy_ref, ewh_ref, boxes_hbm, conf_hbm, cls_hbm,
           *, B, nA, nCp5, nC, P, H, W, aw, ah):
    bpb = _BPB
    pipe = pltpu.emit_pipeline(
        functools.partial(_body, ecls_ref[...], exy_ref[...], ewh_ref[...],
                          W=W, aw=aw, ah=ah),
        grid=(B // bpb,),
        in_specs=[pl.BlockSpec((bpb, nA, nCp5, P), lambda b: (b, 0, 0, 0))],
        out_specs=[
            pl.BlockSpec((bpb, nA, H, W, 4), lambda b: (b, 0, 0, 0, 0)),
            pl.BlockSpec((bpb, nA, 1, P), lambda b: (b, 0, 0, 0)),
            pl.BlockSpec((bpb, nA, H, W, nC), lambda b: (b, 0, 0, 0, 0)),
        ],
    )
    pipe(x_hbm, boxes_hbm, conf_hbm, cls_hbm)


def kernel(x):
    B, C, H, W = x.shape
    nA = _NA
    nCp5 = C // nA
    nC = nCp5 - 5
    P = H * W
    xr = x.reshape(B, nA, nCp5, P)
    aw = tuple(float(a0) * H for (a0, _) in _ANCHORS)
    ah = tuple(float(a1) * W for (_, a1) in _ANCHORS)
    e_cls_np = np.zeros((nCp5, nC), np.float32)
    e_cls_np[5:, :] = np.eye(nC, dtype=np.float32)
    e_cls = jnp.asarray(e_cls_np)
    e_xy_np = np.zeros((2, 4), np.float32)
    e_xy_np[0, 0] = 1.0 / H
    e_xy_np[1, 1] = 1.0 / W
    e_xy = jnp.asarray(e_xy_np)
    e_wh_np = np.zeros((2, 4), np.float32)
    e_wh_np[0, 2] = 1.0 / H
    e_wh_np[1, 3] = 1.0 / W
    e_wh = jnp.asarray(e_wh_np)
    boxes, conf, cls_ = pl.pallas_call(
        functools.partial(_outer, B=B, nA=nA, nCp5=nCp5, nC=nC, P=P,
                          H=H, W=W, aw=aw, ah=ah),
        in_specs=[pl.BlockSpec(memory_space=pl.ANY),
                  pl.BlockSpec(memory_space=pltpu.MemorySpace.VMEM),
                  pl.BlockSpec(memory_space=pltpu.MemorySpace.VMEM),
                  pl.BlockSpec(memory_space=pltpu.MemorySpace.VMEM)],
        out_specs=(
            pl.BlockSpec(memory_space=pl.ANY),
            pl.BlockSpec(memory_space=pl.ANY),
            pl.BlockSpec(memory_space=pl.ANY),
        ),
        out_shape=(
            jax.ShapeDtypeStruct((B, nA, H, W, 4), jnp.float32),
            jax.ShapeDtypeStruct((B, nA, 1, P), jnp.float32),
            jax.ShapeDtypeStruct((B, nA, H, W, nC), jnp.float32),
        ),
        compiler_params=pltpu.CompilerParams(needs_layout_passes=True),
    )(xr, e_cls, e_xy, e_wh)
    return (boxes, conf.reshape(B, nA, H, W), cls_)


# speedup vs baseline: 1.0029x; 1.0029x over previous
import functools

import jax
import jax.numpy as jnp
import numpy as np
from jax.experimental import pallas as pl
from jax.experimental.pallas import tpu as pltpu

_ANCHORS = ((0.28, 0.22), (0.38, 0.48), (0.9, 0.78))
_NA = 3
_BPB = 4


def _body(ecls, exy, ewh, x_ref, boxes_ref, conf_ref, cls_ref, *, W, aw, ah):
    s = x_ref[...]
    P = s.shape[3]
    sig = 1.0 / (1.0 + jnp.exp(-s))

    t_cls = jax.lax.dot_general(
        sig, ecls, (((2,), (0,)), ((), ())),
        preferred_element_type=jnp.float32)
    cls_ref[...] = t_cls.reshape(cls_ref.shape)

    conf_ref[...] = sig[:, :, 4:5, :]

    lan = jax.lax.broadcasted_iota(jnp.int32, (2, P), 1)
    rid = jax.lax.broadcasted_iota(jnp.int32, (2, P), 0)
    off = jnp.where(rid == 0, lan // W, lan % W).astype(jnp.float32)
    xy = sig[:, :, 0:2, :] + off[None, None]
    ex = jnp.exp(s[:, :, 2:4, :])
    aid = jax.lax.broadcasted_iota(jnp.int32, ex.shape, 1)
    aw_v = jnp.where(aid == 0, aw[0], jnp.where(aid == 1, aw[1], aw[2]))
    ah_v = jnp.where(aid == 0, ah[0], jnp.where(aid == 1, ah[1], ah[2]))
    rid2 = jax.lax.broadcasted_iota(jnp.int32, ex.shape, 2)
    wh = ex * jnp.where(rid2 == 0, aw_v, ah_v)
    t_box = (
        jax.lax.dot_general(xy, exy, (((2,), (0,)), ((), ())),
                            preferred_element_type=jnp.float32)
        + jax.lax.dot_general(wh, ewh, (((2,), (0,)), ((), ())),
                              preferred_element_type=jnp.float32))
    boxes_ref[...] = t_box.reshape(boxes_ref.shape)


def _outer(x_hbm, ecls_ref, exy_ref, ewh_ref, boxes_hbm, conf_hbm, cls_hbm,
           *, B, nA, nCp5, nC, P, H, W, aw, ah):
    bpb = _BPB
    pipe = pltpu.emit_pipeline(
        functools.partial(_body, ecls_ref[...], exy_ref[...], ewh_ref[...],
                          W=W, aw=aw, ah=ah),
        grid=(B // bpb,),
        in_specs=[pl.BlockSpec((bpb, nA, nCp5, P), lambda b: (b, 0, 0, 0))],
        out_specs=[
            pl.BlockSpec((bpb, nA, H, W, 4), lambda b: (b, 0, 0, 0, 0)),
            pl.BlockSpec((bpb, nA, 1, P), lambda b: (b, 0, 0, 0)),
            pl.BlockSpec((bpb, nA, H, W, nC), lambda b: (b, 0, 0, 0, 0)),
        ],
    )
    pipe(x_hbm, boxes_hbm, conf_hbm, cls_hbm)


def kernel(x):
    B, C, H, W = x.shape
    nA = _NA
    nCp5 = C // nA
    nC = nCp5 - 5
    P = H * W
    xr = x.reshape(B, nA, nCp5, P)
    aw = tuple(float(a0) * H for (a0, _) in _ANCHORS)
    ah = tuple(float(a1) * W for (_, a1) in _ANCHORS)
    e_cls_np = np.zeros((nCp5, nC), np.float32)
    e_cls_np[5:, :] = np.eye(nC, dtype=np.float32)
    e_cls = jnp.asarray(e_cls_np)
    e_xy_np = np.zeros((2, 4), np.float32)
    e_xy_np[0, 0] = 1.0 / H
    e_xy_np[1, 1] = 1.0 / W
    e_xy = jnp.asarray(e_xy_np)
    e_wh_np = np.zeros((2, 4), np.float32)
    e_wh_np[0, 2] = 1.0 / H
    e_wh_np[1, 3] = 1.0 / W
    e_wh = jnp.asarray(e_wh_np)
    boxes, conf, cls_ = pl.pallas_call(
        functools.partial(_outer, B=B, nA=nA, nCp5=nCp5, nC=nC, P=P,
                          H=H, W=W, aw=aw, ah=ah),
        in_specs=[pl.BlockSpec(memory_space=pl.ANY),
                  pl.BlockSpec(memory_space=pltpu.MemorySpace.VMEM),
                  pl.BlockSpec(memory_space=pltpu.MemorySpace.VMEM),
                  pl.BlockSpec(memory_space=pltpu.MemorySpace.VMEM)],
        out_specs=(
            pl.BlockSpec(memory_space=pl.ANY),
            pl.BlockSpec(memory_space=pl.ANY),
            pl.BlockSpec(memory_space=pl.ANY),
        ),
        out_shape=(
            jax.ShapeDtypeStruct((B, nA, H, W, 4), jnp.float32),
            jax.ShapeDtypeStruct((B, nA, 1, P), jnp.float32),
            jax.ShapeDtypeStruct((B, nA, H, W, nC), jnp.float32),
        ),
        compiler_params=pltpu.CompilerParams(allow_input_fusion=[True, False, False, False]),
    )(xr, e_cls, e_xy, e_wh)
    return (boxes, conf.reshape(B, nA, H, W), cls_)


# speedup vs baseline: 1.0265x; 1.0235x over previous
import functools

import jax
import jax.numpy as jnp
from jax.experimental import pallas as pl

_ANCHORS = ((0.28, 0.22), (0.38, 0.48), (0.9, 0.78))
_NA = 3
_BPB = 4


def _yolo_kernel(x_ref, boxes_ref, conf_ref, cls_ref, *, H, W, aw, ah):
    s = x_ref[...]
    hd = s[:, :, 0:4, :]
    shp = hd.shape
    aid = jax.lax.broadcasted_iota(jnp.int32, shp, 1)
    rid = jax.lax.broadcasted_iota(jnp.int32, shp, 2)
    lan = jax.lax.broadcasted_iota(jnp.int32, shp, 3)
    gx = (lan // W).astype(jnp.float32)
    gy = (lan % W).astype(jnp.float32)
    off = jnp.where(rid == 0, gx, jnp.where(rid == 1, gy, 0.0))
    aw_v = jnp.where(aid == 0, aw[0], jnp.where(aid == 1, aw[1], aw[2]))
    ah_v = jnp.where(aid == 0, ah[0], jnp.where(aid == 1, ah[1], ah[2]))
    anch = jnp.where(rid == 2, aw_v, ah_v)
    inv = jnp.where(rid % 2 == 0, 1.0 / H, 1.0 / W).astype(jnp.float32)
    dec = jnp.where(rid < 2, jax.nn.sigmoid(hd) + off, jnp.exp(hd) * anch)
    boxes_ref[...] = jnp.transpose(dec * inv, (0, 1, 3, 2))
    conf_ref[...] = jax.nn.sigmoid(s[:, :, 4:5, :])
    cls_ref[...] = jnp.transpose(jax.nn.sigmoid(s[:, :, 5:, :]), (0, 1, 3, 2))


def kernel(x):
    B, C, H, W = x.shape
    nA = _NA
    nCp5 = C // nA
    nC = nCp5 - 5
    P = H * W
    bpb = _BPB
    xr = x.reshape(B, nA, nCp5, P)
    aw = tuple(float(a0) * H for (a0, _) in _ANCHORS)
    ah = tuple(float(a1) * W for (_, a1) in _ANCHORS)
    out_shapes = (
        jax.ShapeDtypeStruct((B, nA, P, 4), jnp.float32),
        jax.ShapeDtypeStruct((B, nA, 1, P), jnp.float32),
        jax.ShapeDtypeStruct((B, nA, P, nC), jnp.float32),
    )
    boxes, conf, cls_ = pl.pallas_call(
        functools.partial(_yolo_kernel, H=H, W=W, aw=aw, ah=ah),
        grid=(B // bpb,),
        in_specs=[pl.BlockSpec((bpb, nA, nCp5, P), lambda b: (b, 0, 0, 0))],
        out_specs=(
            pl.BlockSpec((bpb, nA, P, 4), lambda b: (b, 0, 0, 0)),
            pl.BlockSpec((bpb, nA, 1, P), lambda b: (b, 0, 0, 0)),
            pl.BlockSpec((bpb, nA, P, nC), lambda b: (b, 0, 0, 0)),
        ),
        out_shape=out_shapes,
    )(xr)
    return (boxes.reshape(B, nA, H, W, 4),
            conf.reshape(B, nA, H, W),
            cls_.reshape(B, nA, H, W, nC))


# speedup vs baseline: 1.0307x; 1.0041x over previous
import functools

import jax
import jax.numpy as jnp
from jax.experimental import pallas as pl

_ANCHORS = ((0.28, 0.22), (0.38, 0.48), (0.9, 0.78))
_NA = 3
_BPB = 2


def _yolo_kernel(x_ref, boxes_ref, conf_ref, cls_ref, *, H, W, aw, ah):
    s = x_ref[...]
    hd = s[:, :, 0:4, :]
    shp = hd.shape
    aid = jax.lax.broadcasted_iota(jnp.int32, shp, 1)
    rid = jax.lax.broadcasted_iota(jnp.int32, shp, 2)
    lan = jax.lax.broadcasted_iota(jnp.int32, shp, 3)
    gx = (lan // W).astype(jnp.float32)
    gy = (lan % W).astype(jnp.float32)
    off = jnp.where(rid == 0, gx, jnp.where(rid == 1, gy, 0.0))
    aw_v = jnp.where(aid == 0, aw[0], jnp.where(aid == 1, aw[1], aw[2]))
    ah_v = jnp.where(aid == 0, ah[0], jnp.where(aid == 1, ah[1], ah[2]))
    anch = jnp.where(rid == 2, aw_v, ah_v)
    inv = jnp.where(rid % 2 == 0, 1.0 / H, 1.0 / W).astype(jnp.float32)
    dec = jnp.where(rid < 2, jax.nn.sigmoid(hd) + off, jnp.exp(hd) * anch)
    boxes_ref[...] = jnp.transpose(dec * inv, (0, 1, 3, 2))
    conf_ref[...] = jax.nn.sigmoid(s[:, :, 4:5, :])
    cls_ref[...] = jnp.transpose(jax.nn.sigmoid(s[:, :, 5:, :]), (0, 1, 3, 2))


def kernel(x):
    B, C, H, W = x.shape
    nA = _NA
    nCp5 = C // nA
    nC = nCp5 - 5
    P = H * W
    bpb = _BPB
    xr = x.reshape(B, nA, nCp5, P)
    aw = tuple(float(a0) * H for (a0, _) in _ANCHORS)
    ah = tuple(float(a1) * W for (_, a1) in _ANCHORS)
    out_shapes = (
        jax.ShapeDtypeStruct((B, nA, P, 4), jnp.float32),
        jax.ShapeDtypeStruct((B, nA, 1, P), jnp.float32),
        jax.ShapeDtypeStruct((B, nA, P, nC), jnp.float32),
    )
    boxes, conf, cls_ = pl.pallas_call(
        functools.partial(_yolo_kernel, H=H, W=W, aw=aw, ah=ah),
        grid=(B // bpb,),
        in_specs=[pl.BlockSpec((bpb, nA, nCp5, P), lambda b: (b, 0, 0, 0))],
        out_specs=(
            pl.BlockSpec((bpb, nA, P, 4), lambda b: (b, 0, 0, 0)),
            pl.BlockSpec((bpb, nA, 1, P), lambda b: (b, 0, 0, 0)),
            pl.BlockSpec((bpb, nA, P, nC), lambda b: (b, 0, 0, 0)),
        ),
        out_shape=out_shapes,
    )(xr)
    return (boxes.reshape(B, nA, H, W, 4),
            conf.reshape(B, nA, H, W),
            cls_.reshape(B, nA, H, W, nC))
